# baseline (speedup 1.0000x reference)
import functools
import math

import jax
import jax.numpy as jnp
from jax import lax
from jax.experimental import pallas as pl
from jax.experimental.pallas import tpu as pltpu
from jax.experimental.pallas import tpu_sc as plsc

D_MODEL = 64
SCALE = math.sqrt(D_MODEL)

_info = plsc.get_sparse_core_info()
_NC = _info.num_cores
_NS = _info.num_subcores
_L = _info.num_lanes
_NW = _NC * _NS


@functools.partial(jax.jit, static_argnums=(2, 3))
def _gather_scale(idx_flat, lut, B, C):
    b_per_w = B // _NW
    n_chunks = b_per_w // C
    mesh = plsc.VectorSubcoreMesh(core_axis_name="c", subcore_axis_name="s")

    @functools.partial(
        pl.kernel,
        mesh=mesh,
        out_type=jax.ShapeDtypeStruct((B, D_MODEL), jnp.float32),
        scratch_types=[
            pltpu.VMEM((C,), jnp.int32),
            pltpu.VMEM((C, D_MODEL), jnp.float32),
            pltpu.SemaphoreType.DMA,
        ],
        compiler_params=pltpu.CompilerParams(use_tc_tiling_on_sc=False),
    )
    def k(idx_hbm, table_hbm, out_hbm, idx_v, rows_v, sem):
        wid = lax.axis_index("s") * _NC + lax.axis_index("c")
        base = wid * b_per_w

        def chunk_body(i, carry):
            off = base + i * C
            pltpu.sync_copy(idx_hbm.at[pl.ds(off, C)], idx_v)
            pltpu.async_copy(table_hbm.at[idx_v], rows_v, sem).wait()

            def scale_row(r, carry2):
                for j in range(D_MODEL // _L):
                    sl = pl.ds(j * _L, _L)
                    rows_v[r, sl] = rows_v[r, sl] * SCALE
                return carry2

            lax.fori_loop(0, C, scale_row, 0, unroll=2)
            pltpu.sync_copy(rows_v, out_hbm.at[pl.ds(off, C)])
            return carry

        lax.fori_loop(0, n_chunks, chunk_body, 0)

    return k(idx_flat, lut)


def kernel(x, lut):
    B = x.shape[0] * x.shape[1]
    idx_flat = x.reshape(B).astype(jnp.int32)
    out = _gather_scale(idx_flat, lut, B, 128)
    return out.reshape(x.shape[0], x.shape[1], D_MODEL)


# speedup vs baseline: 1.1107x; 1.1107x over previous
import functools
import math

import jax
import jax.numpy as jnp
from jax import lax
from jax.experimental import pallas as pl
from jax.experimental.pallas import tpu as pltpu
from jax.experimental.pallas import tpu_sc as plsc

D_MODEL = 64
SCALE = math.sqrt(D_MODEL)

_info = plsc.get_sparse_core_info()
_NC = _info.num_cores
_NS = _info.num_subcores
_L = _info.num_lanes
_NW = _NC * _NS

_NBUF = 4
_WBUF = 2


@functools.partial(jax.jit, static_argnums=(2, 3))
def _gather_scale(idx_flat, lut, B, C):
    b_per_w = B // _NW
    n_chunks = b_per_w // C
    assert n_chunks % _NBUF == 0 and n_chunks >= 2 * _NBUF
    mesh = plsc.VectorSubcoreMesh(core_axis_name="c", subcore_axis_name="s")

    @functools.partial(
        pl.kernel,
        mesh=mesh,
        out_type=jax.ShapeDtypeStruct((B, D_MODEL), jnp.float32),
        scratch_types=(
            [pltpu.VMEM((b_per_w,), jnp.int32),
             pltpu.VMEM((_NBUF, C, D_MODEL), jnp.float32),
             pltpu.VMEM((_WBUF, C, D_MODEL), jnp.float32)]
            + [pltpu.SemaphoreType.DMA] * (_NBUF + _WBUF)
        ),
        compiler_params=pltpu.CompilerParams(use_tc_tiling_on_sc=False),
    )
    def k(idx_hbm, table_hbm, out_hbm, idx_all, grows, wrows, *sems):
        gsems = sems[:_NBUF]
        wsems = sems[_NBUF:]
        wid = lax.axis_index("s") * _NC + lax.axis_index("c")
        base = wid * b_per_w
        pltpu.sync_copy(idx_hbm.at[pl.ds(base, b_per_w)], idx_all)

        def start_gather(i_chunk, gb):
            pltpu.async_copy(
                table_hbm.at[idx_all.at[pl.ds(i_chunk * C, C)]],
                grows.at[gb], gsems[gb])

        def wait_gather(gb):
            pltpu.make_async_copy(
                table_hbm.at[pl.ds(0, C)], grows.at[gb], gsems[gb]).wait()

        def start_wb(i_chunk, wb):
            pltpu.async_copy(
                wrows.at[wb], out_hbm.at[pl.ds(base + i_chunk * C, C)],
                wsems[wb])

        def wait_wb(wb):
            pltpu.make_async_copy(
                wrows.at[wb], out_hbm.at[pl.ds(base, C)], wsems[wb]).wait()

        def scale(gb, wb):
            def row_body(r, carry):
                for j in range(D_MODEL // _L):
                    sl = pl.ds(j * _L, _L)
                    wrows[wb, r, sl] = grows[gb, r, sl] * SCALE
                return carry
            lax.fori_loop(0, C, row_body, 0, unroll=4)

        for b in range(_NBUF):
            start_gather(b, b)

        for b in range(_NBUF):
            wait_gather(b)
            if b >= _WBUF:
                wait_wb(b % _WBUF)
            scale(b, b % _WBUF)
            start_gather(b + _NBUF, b)
            start_wb(b, b % _WBUF)

        def outer(g, carry):
            for b in range(_NBUF):
                i = g * _NBUF + b
                wait_gather(b)
                wait_wb(b % _WBUF)
                scale(b, b % _WBUF)
                start_gather(i + _NBUF, b)
                start_wb(i, b % _WBUF)
            return carry

        lax.fori_loop(1, n_chunks // _NBUF - 1, outer, 0)

        for b in range(_NBUF):
            i = n_chunks - _NBUF + b
            wait_gather(b)
            wait_wb(b % _WBUF)
            scale(b, b % _WBUF)
            start_wb(i, b % _WBUF)

        for wb in range(_WBUF):
            wait_wb(wb)

    return k(idx_flat, lut)


def kernel(x, lut):
    B = x.shape[0] * x.shape[1]
    idx_flat = x.reshape(B).astype(jnp.int32)
    out = _gather_scale(idx_flat, lut, B, 128)
    return out.reshape(x.shape[0], x.shape[1], D_MODEL)
